# Initial kernel scaffold; baseline (speedup 1.0000x reference)
import jax
import jax.numpy as jnp
from jax.experimental import pallas as pl


def kernel(x, conv_w, conv_b, w1, b1, w2, b2, w3, b3):
    raise NotImplementedError("write your pallas kernel here")


# speedup vs baseline: 7.8243x; 7.8243x over previous
import functools

import jax
import jax.numpy as jnp
from jax import lax
from jax.experimental import pallas as pl
from jax.experimental.pallas import tpu as pltpu

B, C, N = 8, 1024, 4096
J, R = 64, 32
NBLK = 512
NBLKS = N // NBLK


def _conv_topk_body(x_ref, w_ref, b_ref, out_ref, y_sc):
    n = pl.program_id(1)
    yb = jnp.dot(w_ref[...], x_ref[0], preferred_element_type=jnp.float32)
    y_sc[:, pl.ds(n * NBLK, NBLK)] = yb + b_ref[...]

    @pl.when(n == NBLKS - 1)
    def _():
        y = y_sc[...]
        iota = lax.broadcasted_iota(jnp.int32, (J, N), 1)
        cols = [None] * (2 * R)
        work = y
        for i in range(R):
            m = jnp.max(work, axis=1, keepdims=True)
            idx = jnp.min(jnp.where(work == m, iota, N), axis=1, keepdims=True)
            cols[i] = m
            work = jnp.where(iota == idx, -jnp.inf, work)
        work = y
        for i in range(R):
            m = jnp.min(work, axis=1, keepdims=True)
            idx = jnp.min(jnp.where(work == m, iota, N), axis=1, keepdims=True)
            cols[2 * R - 1 - i] = m
            work = jnp.where(iota == idx, jnp.inf, work)
        out_ref[0] = jnp.concatenate(cols, axis=1)


def _mlp_body(f_ref, w1_ref, b1_ref, w2_ref, b2_ref, w3_ref, b3_ref, out_ref):
    h = jax.nn.sigmoid(
        jnp.dot(f_ref[...], w1_ref[...], preferred_element_type=jnp.float32)
        + b1_ref[...]
    )
    h = jax.nn.sigmoid(
        jnp.dot(h, w2_ref[...], preferred_element_type=jnp.float32) + b2_ref[...]
    )
    out_ref[...] = (
        jnp.dot(h, w3_ref[...], preferred_element_type=jnp.float32) + b3_ref[...]
    )


@jax.jit
def kernel(x, conv_w, conv_b, w1, b1, w2, b2, w3, b3):
    topk = pl.pallas_call(
        _conv_topk_body,
        grid=(B, NBLKS),
        in_specs=[
            pl.BlockSpec((1, C, NBLK), lambda b, n: (b, 0, n)),
            pl.BlockSpec((J, C), lambda b, n: (0, 0)),
            pl.BlockSpec((J, 1), lambda b, n: (0, 0)),
        ],
        out_specs=pl.BlockSpec((1, J, 2 * R), lambda b, n: (b, 0, 0)),
        out_shape=jax.ShapeDtypeStruct((B, J, 2 * R), jnp.float32),
        scratch_shapes=[pltpu.VMEM((J, N), jnp.float32)],
        compiler_params=pltpu.CompilerParams(
            dimension_semantics=("arbitrary", "arbitrary"),
        ),
    )(x, conv_w, conv_b.reshape(J, 1))

    flat = topk.reshape(B, 2 * R * J)
    logits = pl.pallas_call(
        _mlp_body,
        in_specs=[pl.BlockSpec(a.shape, lambda: (0,) * a.ndim) for a in
                  (flat, w1, b1.reshape(1, -1), w2, b2.reshape(1, -1),
                   w3, b3.reshape(1, -1))],
        out_specs=pl.BlockSpec((B, 2), lambda: (0, 0)),
        out_shape=jax.ShapeDtypeStruct((B, 2), jnp.float32),
    )(flat, w1, b1.reshape(1, -1), w2, b2.reshape(1, -1), w3, b3.reshape(1, -1))
    return logits
